# Initial kernel scaffold; baseline (speedup 1.0000x reference)
import jax
import jax.numpy as jnp
from jax.experimental import pallas as pl


def kernel(tokens, token_table, position_embedding):
    raise NotImplementedError("write your pallas kernel here")


# speedup vs baseline: 2.4919x; 2.4919x over previous
import functools

import jax
import jax.numpy as jnp
from jax import lax
from jax.experimental import pallas as pl
from jax.experimental.pallas import tpu as pltpu
from jax.experimental.pallas import tpu_sc as plsc

B = 4096
T = 200
D = 64
NC = 2
NS = 16
NW = NC * NS

CHUNK = 128
TOTAL_ROWS = B * T
ROWS_PER_W = TOTAL_ROWS // NW
CHUNKS_PER_W = ROWS_PER_W // CHUNK
LANES = 16
GROUPS = D // LANES


@functools.partial(
    pl.kernel,
    out_type=jax.ShapeDtypeStruct((TOTAL_ROWS, D), jnp.float32),
    mesh=plsc.VectorSubcoreMesh(core_axis_name="c", subcore_axis_name="s"),
    scratch_types=[
        pltpu.VMEM((CHUNKS_PER_W, CHUNK), jnp.int32),
        pltpu.VMEM((T, D), jnp.float32),
        pltpu.VMEM((CHUNK, D), jnp.float32),
        pltpu.SemaphoreType.DMA,
    ],
    compiler_params=pltpu.CompilerParams(use_tc_tiling_on_sc=False),
)
def _emb_kernel(idx_hbm, table_hbm, pos_hbm, out_hbm, idx_v, pos_v, buf, sem):
    cid = lax.axis_index("c")
    sid = lax.axis_index("s")
    wid = sid * NC + cid
    chunk0 = wid * CHUNKS_PER_W
    row0 = wid * ROWS_PER_W

    pltpu.sync_copy(idx_hbm.at[pl.ds(chunk0, CHUNKS_PER_W)], idx_v)
    pltpu.sync_copy(pos_hbm, pos_v)

    def chunk_body(ci, carry):
        pltpu.async_copy(table_hbm.at[idx_v.at[ci]], buf, sem).wait()

        p0 = lax.rem(row0 + ci * CHUNK, T)

        def row_body(r, pr):
            for g in range(GROUPS):
                pv = pos_v[pr, pl.ds(g * LANES, LANES)]
                plsc.addupdate(buf.at[r, pl.ds(g * LANES, LANES)], pv)
            nxt = pr + 1
            return lax.select(nxt == T, 0, nxt)

        lax.fori_loop(0, CHUNK, row_body, p0, unroll=2)

        pltpu.sync_copy(buf, out_hbm.at[pl.ds(row0 + ci * CHUNK, CHUNK)])
        return carry

    lax.fori_loop(0, CHUNKS_PER_W, chunk_body, 0)


def kernel(tokens, token_table, position_embedding):
    idx = tokens.reshape(TOTAL_ROWS // CHUNK, CHUNK).astype(jnp.int32)
    out = _emb_kernel(idx, token_table, position_embedding)
    return out.reshape(B, T, D)


# speedup vs baseline: 3.0198x; 1.2118x over previous
import functools

import jax
import jax.numpy as jnp
from jax import lax
from jax.experimental import pallas as pl
from jax.experimental.pallas import tpu as pltpu
from jax.experimental.pallas import tpu_sc as plsc

B = 4096
T = 200
D = 64
NC = 2
NS = 16
NW = NC * NS

CHUNK = 128
TOTAL_ROWS = B * T
ROWS_PER_W = TOTAL_ROWS // NW
CHUNKS_PER_W = ROWS_PER_W // CHUNK
LANES = 16
GROUPS = D // LANES


@functools.partial(
    pl.kernel,
    out_type=jax.ShapeDtypeStruct((TOTAL_ROWS, D), jnp.float32),
    mesh=plsc.VectorSubcoreMesh(core_axis_name="c", subcore_axis_name="s"),
    scratch_types=[
        pltpu.VMEM((CHUNKS_PER_W, CHUNK), jnp.int32),
        pltpu.VMEM((T, D), jnp.float32),
        pltpu.VMEM((CHUNK, D), jnp.float32),
        pltpu.VMEM((CHUNK, D), jnp.float32),
        pltpu.SemaphoreType.DMA,
        pltpu.SemaphoreType.DMA,
        pltpu.SemaphoreType.DMA,
        pltpu.SemaphoreType.DMA,
    ],
    compiler_params=pltpu.CompilerParams(use_tc_tiling_on_sc=False),
)
def _emb_kernel(idx_hbm, table_hbm, pos_hbm, out_hbm, idx_v, pos_v,
                buf0, buf1, gsem0, gsem1, ssem0, ssem1):
    cid = lax.axis_index("c")
    sid = lax.axis_index("s")
    wid = sid * NC + cid
    chunk0 = wid * CHUNKS_PER_W
    row0 = wid * ROWS_PER_W
    bufs = (buf0, buf1)
    gsems = (gsem0, gsem1)
    ssems = (ssem0, ssem1)

    pltpu.sync_copy(idx_hbm.at[pl.ds(chunk0, CHUNKS_PER_W)], idx_v)
    pltpu.sync_copy(pos_hbm, pos_v)

    def gather(ci, b):
        pltpu.make_async_copy(
            table_hbm.at[idx_v.at[ci]], bufs[b], gsems[b]).start()

    def gather_wait(ci, b):
        pltpu.make_async_copy(
            table_hbm.at[idx_v.at[ci]], bufs[b], gsems[b]).wait()

    def store(ci, b):
        pltpu.make_async_copy(
            bufs[b], out_hbm.at[pl.ds(row0 + ci * CHUNK, CHUNK)],
            ssems[b]).start()

    def store_wait(ci, b):
        pltpu.make_async_copy(
            bufs[b], out_hbm.at[pl.ds(row0 + ci * CHUNK, CHUNK)],
            ssems[b]).wait()

    def add_pos(ci, b):
        p0 = lax.rem(row0 + ci * CHUNK, T)
        buf = bufs[b]

        def row_body(r, pr):
            for g in range(GROUPS):
                pv = pos_v[pr, pl.ds(g * LANES, LANES)]
                plsc.addupdate(buf.at[r, pl.ds(g * LANES, LANES)], pv)
            nxt = pr + 1
            return lax.select(nxt == T, 0, nxt)

        lax.fori_loop(0, CHUNK, row_body, p0, unroll=2)

    gather(0, 0)

    def outer(oi, carry):
        for b in range(2):
            ci = oi * 2 + b
            ob = 1 - b

            @pl.when(ci + 1 < CHUNKS_PER_W)
            def _fire_next():
                @pl.when(ci >= 1)
                def _drain_store():
                    store_wait(ci - 1, ob)
                gather(ci + 1, ob)

            gather_wait(ci, b)
            add_pos(ci, b)
            store(ci, b)
        return carry

    lax.fori_loop(0, CHUNKS_PER_W // 2, outer, 0)
    store_wait(CHUNKS_PER_W - 2, 0)
    store_wait(CHUNKS_PER_W - 1, 1)


def kernel(tokens, token_table, position_embedding):
    idx = tokens.reshape(TOTAL_ROWS // CHUNK, CHUNK).astype(jnp.int32)
    out = _emb_kernel(idx, token_table, position_embedding)
    return out.reshape(B, T, D)


# speedup vs baseline: 3.8564x; 1.2770x over previous
import functools

import jax
import jax.numpy as jnp
from jax import lax
from jax.experimental import pallas as pl
from jax.experimental.pallas import tpu as pltpu
from jax.experimental.pallas import tpu_sc as plsc

B = 4096
T = 200
D = 64
NC = 2
NS = 16
NW = NC * NS

CHUNK = 128
TOTAL_ROWS = B * T
ROWS_PER_W = TOTAL_ROWS // NW
CHUNKS_PER_W = ROWS_PER_W // CHUNK
LANES = 16
GROUPS = D // LANES


@functools.partial(
    pl.kernel,
    out_type=jax.ShapeDtypeStruct((TOTAL_ROWS, D), jnp.float32),
    mesh=plsc.VectorSubcoreMesh(core_axis_name="c", subcore_axis_name="s"),
    scratch_types=[
        pltpu.VMEM((CHUNKS_PER_W, CHUNK), jnp.int32),
        pltpu.VMEM((T, D), jnp.float32),
        pltpu.VMEM((CHUNK, D), jnp.float32),
        pltpu.VMEM((CHUNK, D), jnp.float32),
        pltpu.SemaphoreType.DMA,
        pltpu.SemaphoreType.DMA,
        pltpu.SemaphoreType.DMA,
        pltpu.SemaphoreType.DMA,
    ],
    compiler_params=pltpu.CompilerParams(use_tc_tiling_on_sc=False),
)
def _emb_kernel(idx_hbm, table_hbm, pos_hbm, out_hbm, idx_v, pos_v,
                buf0, buf1, gsem0, gsem1, ssem0, ssem1):
    cid = lax.axis_index("c")
    sid = lax.axis_index("s")
    wid = sid * NC + cid
    chunk0 = wid * CHUNKS_PER_W
    row0 = wid * ROWS_PER_W
    bufs = (buf0, buf1)
    gsems = (gsem0, gsem1)
    ssems = (ssem0, ssem1)

    pltpu.sync_copy(idx_hbm.at[pl.ds(chunk0, CHUNKS_PER_W)], idx_v)
    pltpu.sync_copy(pos_hbm, pos_v)

    def gather(ci, b):
        pltpu.make_async_copy(
            table_hbm.at[idx_v.at[ci]], bufs[b], gsems[b]).start()

    def gather_wait(ci, b):
        pltpu.make_async_copy(
            table_hbm.at[idx_v.at[ci]], bufs[b], gsems[b]).wait()

    def store(ci, b):
        pltpu.make_async_copy(
            bufs[b], out_hbm.at[pl.ds(row0 + ci * CHUNK, CHUNK)],
            ssems[b]).start()

    def store_wait(ci, b):
        pltpu.make_async_copy(
            bufs[b], out_hbm.at[pl.ds(row0 + ci * CHUNK, CHUNK)],
            ssems[b]).wait()

    def add_pos(ci, b):
        p0 = lax.rem(row0 + ci * CHUNK, T)
        n1 = lax.min(CHUNK, T - p0)
        buf = bufs[b]

        def seg(lo, hi, poff):
            @plsc.parallel_loop(lo, hi, unroll=4)
            def _seg(r):
                pr = poff + r
                for g in range(GROUPS):
                    pv = pos_v[pr, pl.ds(g * LANES, LANES)]
                    plsc.addupdate(buf.at[r, pl.ds(g * LANES, LANES)], pv)

        seg(0, n1, p0)
        seg(n1, CHUNK, -n1)

    gather(0, 0)

    def outer(oi, carry):
        for b in range(2):
            ci = oi * 2 + b
            ob = 1 - b

            @pl.when(ci + 1 < CHUNKS_PER_W)
            def _fire_next():
                @pl.when(ci >= 1)
                def _drain_store():
                    store_wait(ci - 1, ob)
                gather(ci + 1, ob)

            gather_wait(ci, b)
            add_pos(ci, b)
            store(ci, b)
        return carry

    lax.fori_loop(0, CHUNKS_PER_W // 2, outer, 0)
    store_wait(CHUNKS_PER_W - 2, 0)
    store_wait(CHUNKS_PER_W - 1, 1)


def kernel(tokens, token_table, position_embedding):
    idx = tokens.reshape(TOTAL_ROWS // CHUNK, CHUNK).astype(jnp.int32)
    out = _emb_kernel(idx, token_table, position_embedding)
    return out.reshape(B, T, D)


# speedup vs baseline: 12.5910x; 3.2650x over previous
import functools

import jax
import jax.numpy as jnp
from jax import lax
from jax.experimental import pallas as pl
from jax.experimental.pallas import tpu as pltpu
from jax.experimental.pallas import tpu_sc as plsc

B = 4096
T = 200
D = 64
NC = 2
NS = 16
NW = NC * NS

BLK = B // NW
LANES = 16
GROUPS = D // LANES


@functools.partial(
    pl.kernel,
    out_type=jax.ShapeDtypeStruct((T, D // 8, NW, 8 * BLK), jnp.float32),
    mesh=plsc.VectorSubcoreMesh(core_axis_name="c", subcore_axis_name="s"),
    scratch_types=[
        pltpu.VMEM((T, BLK), jnp.int32),
        pltpu.VMEM((T, D), jnp.float32),
        pltpu.VMEM((BLK, D), jnp.float32),
        pltpu.VMEM((BLK, D), jnp.float32),
        pltpu.VMEM((BLK, D), jnp.float32),
        pltpu.VMEM((BLK, D), jnp.float32),
        pltpu.VMEM((BLK, D + 1), jnp.float32),
        pltpu.VMEM((D * BLK,), jnp.float32),
        pltpu.VMEM((D * BLK,), jnp.float32),
        pltpu.SemaphoreType.DMA,
        pltpu.SemaphoreType.DMA,
        pltpu.SemaphoreType.DMA,
        pltpu.SemaphoreType.DMA,
        pltpu.SemaphoreType.DMA,
        pltpu.SemaphoreType.DMA,
    ],
    compiler_params=pltpu.CompilerParams(use_tc_tiling_on_sc=False,
                                         needs_layout_passes=False),
)
def _emb_kernel(idx_hbm, table_hbm, pos_hbm, out_hbm, idx_v, pos_v,
                buf0, buf1, buf2, buf3, pad_v, tile0, tile1,
                gsem0, gsem1, gsem2, gsem3, ssem0, ssem1):
    cid = lax.axis_index("c")
    sid = lax.axis_index("s")
    wid = sid * NC + cid
    bufs = (buf0, buf1, buf2, buf3)
    tiles = (tile0, tile1)
    gsems = (gsem0, gsem1, gsem2, gsem3)
    ssems = (ssem0, ssem1)

    pltpu.sync_copy(idx_hbm.at[:, pl.ds(wid * BLK, BLK)], idx_v)
    pltpu.sync_copy(pos_hbm, pos_v)

    iota = lax.iota(jnp.int32, LANES)
    row_c = [blk * LANES + iota for blk in range(BLK // LANES)]

    def gather(t, b):
        pltpu.make_async_copy(
            table_hbm.at[idx_v.at[t]], bufs[b], gsems[b]).start()

    def gather_wait(t, b):
        pltpu.make_async_copy(
            table_hbm.at[idx_v.at[t]], bufs[b], gsems[b]).wait()

    def store(t, b):
        for dt in range(D // 8):
            pltpu.make_async_copy(
                tiles[b].at[pl.ds(dt * 8 * BLK, 8 * BLK)],
                out_hbm.at[t, dt, wid], ssems[b]).start()

    def store_wait(t, b):
        for dt in range(D // 8):
            pltpu.make_async_copy(
                tiles[b].at[pl.ds(dt * 8 * BLK, 8 * BLK)],
                out_hbm.at[t, dt, wid], ssems[b]).wait()

    def transpose_add(t, b, tb):
        buf = bufs[b]
        tile = tiles[tb]
        pv = [pos_v[t, pl.ds(g * LANES, LANES)] for g in range(GROUPS)]

        @plsc.parallel_loop(0, BLK, unroll=2)
        def _bl(bl):
            for g in range(GROUPS):
                v = buf[bl, pl.ds(g * LANES, LANES)] + pv[g]
                pad_v[bl, pl.ds(g * LANES, LANES)] = v

        @plsc.parallel_loop(0, D, unroll=2)
        def _d(d):
            dv = lax.full((LANES,), d, jnp.int32)
            fbase = (lax.shift_left(lax.shift_right_logical(d, 3), 10)
                     + lax.shift_left(lax.bitwise_and(d, 7), 7))
            for blk in range(BLK // LANES):
                v = plsc.load_gather(pad_v, [row_c[blk], dv])
                tile[pl.ds(fbase + blk * LANES, LANES)] = v

    gather(0, 0)
    gather(1, 1)
    gather(2, 2)

    def outer(oi, carry):
        for b in range(4):
            t = oi * 4 + b
            tb = b & 1

            @pl.when(t + 3 < T)
            def _fire_next():
                gather(t + 3, (b + 3) & 3)

            gather_wait(t, b)

            @pl.when(t >= 2)
            def _drain_store():
                store_wait(t - 2, tb)

            transpose_add(t, b, tb)
            store(t, tb)
        return carry

    lax.fori_loop(0, T // 4, outer, 0)
    store_wait(T - 2, 0)
    store_wait(T - 1, 1)


def kernel(tokens, token_table, position_embedding):
    idx = tokens.T.astype(jnp.int32)
    out4 = _emb_kernel(idx, token_table, position_embedding)
    out = (out4.reshape(T, D // 8, NW, 8, BLK)
           .transpose(2, 4, 0, 1, 3)
           .reshape(B, T, D))
    return out
